# baseline (speedup 1.0000x reference)
import functools

import jax
import jax.numpy as jnp
from jax import lax
from jax.experimental import pallas as pl
from jax.experimental.pallas import tpu as pltpu
from jax.experimental.pallas import tpu_sc as plsc

N = 10000
E = 320000
D = 128
C = 16

NC = 2
NS = 16
NGRP = 4
NSH = 8
TE = 128
NTILE = E // TE


def _tables_kernel(x_ref, w_ref, b_ref, yg_ref):
    x = x_ref[...]
    w = w_ref[...]
    dn = (((1,), (1,)), ((), ()))
    y1 = lax.dot_general(x, w[:, :D], dn, preferred_element_type=jnp.float32)
    y2 = lax.dot_general(x, w[:, D:], dn,
                         preferred_element_type=jnp.float32) + b_ref[...]
    for g in range(NGRP):
        yg_ref[g] = jnp.concatenate(
            [y1[:, 4 * g:4 * g + 4], y2[:, 4 * g:4 * g + 4]], axis=1)


def _edge_kernel(yg_hbm, ei_hbm, out_hbm, tab, idxb, sbuf, sem_t, sem_i, sem_s):
    wid = lax.axis_index("s") * NC + lax.axis_index("c")
    grp = lax.rem(wid, NGRP)
    shard = lax.div(wid, NGRP)
    kmax = lax.div(NTILE - shard + NSH - 1, NSH)
    c_hi = lax.div(grp, 2)
    c_lo = 4 * lax.rem(grp, 2)

    def tidx(k):
        return shard + k * NSH

    def fire_idx(k, b):
        off = tidx(k) * TE
        pltpu.async_copy(ei_hbm.at[pl.ds(0, 2), pl.ds(off, TE)],
                         idxb.at[b], sem_i.at[b])

    def wait_idx(k, b):
        off = tidx(k) * TE
        pltpu.make_async_copy(ei_hbm.at[pl.ds(0, 2), pl.ds(off, TE)],
                              idxb.at[b], sem_i.at[b]).wait()

    def store(k, b):
        pltpu.async_copy(sbuf.at[b], out_hbm.at[c_hi, tidx(k), pl.ds(c_lo, 4)],
                         sem_s.at[b])

    def wait_store(k, b):
        pltpu.make_async_copy(sbuf.at[b],
                              out_hbm.at[c_hi, tidx(k), pl.ds(c_lo, 4)],
                              sem_s.at[b]).wait()

    pltpu.async_copy(yg_hbm.at[grp], tab, sem_t)
    fire_idx(0, 0)
    fire_idx(1, 1)
    pltpu.make_async_copy(yg_hbm.at[grp], tab, sem_t).wait()

    c1_vec = [jnp.full((16,), c, dtype=jnp.int32) for c in range(4)]
    c2_vec = [jnp.full((16,), 4 + c, dtype=jnp.int32) for c in range(4)]

    def body(k, _):
        b3 = lax.rem(k, 3)
        b2 = lax.rem(k, 2)

        wait_idx(k, b3)

        @pl.when(k + 2 < kmax)
        def _():
            fire_idx(k + 2, lax.rem(k + 2, 3))

        @pl.when(k >= 2)
        def _():
            wait_store(k - 2, b2)

        for g16 in range(TE // 16):
            src_v = idxb[b3, 0, pl.ds(16 * g16, 16)]
            dst_v = idxb[b3, 1, pl.ds(16 * g16, 16)]
            for c in range(4):
                v = (plsc.load_gather(tab, [src_v, c1_vec[c]])
                     + plsc.load_gather(tab, [dst_v, c2_vec[c]]))
                sbuf[b2, c, pl.ds(16 * g16, 16)] = v

        store(k, b2)
        return 0

    lax.fori_loop(0, kmax, body, 0)

    wait_store(kmax - 2, lax.rem(kmax - 2, 2))
    wait_store(kmax - 1, lax.rem(kmax - 1, 2))


def kernel(x, edge_index, W, b):
    yg = pl.pallas_call(
        _tables_kernel,
        out_shape=jax.ShapeDtypeStruct((NGRP, N, 8), jnp.float32),
    )(x, W, b.reshape(1, C))

    ei = edge_index.astype(jnp.int32)

    mesh = plsc.VectorSubcoreMesh(core_axis_name="c", subcore_axis_name="s",
                                  num_cores=NC, num_subcores=NS)
    out4 = pl.kernel(
        _edge_kernel,
        out_type=jax.ShapeDtypeStruct((2, NTILE, 8, TE), jnp.float32),
        mesh=mesh,
        scratch_types=[
            pltpu.VMEM((N, 8), jnp.float32),
            pltpu.VMEM((3, 2, TE), jnp.int32),
            pltpu.VMEM((2, 4, TE), jnp.float32),
            pltpu.SemaphoreType.DMA,
            pltpu.SemaphoreType.DMA((3,)),
            pltpu.SemaphoreType.DMA((2,)),
        ],
        compiler_params=pltpu.CompilerParams(use_tc_tiling_on_sc=False,
                                             needs_layout_passes=False),
    )(yg, ei)

    return out4.transpose(0, 2, 1, 3).reshape(C, E).T


# speedup vs baseline: 1.0123x; 1.0123x over previous
import functools

import jax
import jax.numpy as jnp
from jax import lax
from jax.experimental import pallas as pl
from jax.experimental.pallas import tpu as pltpu
from jax.experimental.pallas import tpu_sc as plsc

N = 10000
E = 320000
D = 128
C = 16

NC = 2
NS = 16
NGRP = 4
NSH = 8
TE = 128
NTILE = E // TE
TB = 4
NBLK = NTILE // TB


def _tables_kernel(x_ref, w_ref, b_ref, yg_ref):
    x = x_ref[...]
    w = w_ref[...]
    dn = (((1,), (1,)), ((), ()))
    y1 = lax.dot_general(x, w[:, :D], dn, preferred_element_type=jnp.float32)
    y2 = lax.dot_general(x, w[:, D:], dn,
                         preferred_element_type=jnp.float32) + b_ref[...]
    for g in range(NGRP):
        yg_ref[g] = jnp.concatenate(
            [y1[:, 4 * g:4 * g + 4], y2[:, 4 * g:4 * g + 4]], axis=1)


def _edge_kernel(yg_hbm, ei_hbm, out_hbm, tab, idxb, sbuf, sem_t, sem_i, sem_s):
    wid = lax.axis_index("s") * NC + lax.axis_index("c")
    grp = lax.rem(wid, NGRP)
    shard = lax.div(wid, NGRP)
    kmax = lax.div(NBLK - shard + NSH - 1, NSH)
    c_hi = lax.div(grp, 2)
    c_lo = 4 * lax.rem(grp, 2)

    def t0(k):
        return (shard + k * NSH) * TB

    def fire_idx(k, b):
        off = t0(k) * TE
        pltpu.async_copy(ei_hbm.at[pl.ds(0, 2), pl.ds(off, TB * TE)],
                         idxb.at[b], sem_i.at[b])

    def wait_idx(k, b):
        off = t0(k) * TE
        pltpu.make_async_copy(ei_hbm.at[pl.ds(0, 2), pl.ds(off, TB * TE)],
                              idxb.at[b], sem_i.at[b]).wait()

    def store(k, b):
        pltpu.async_copy(sbuf.at[b],
                         out_hbm.at[c_hi, pl.ds(t0(k), TB), pl.ds(c_lo, 4)],
                         sem_s.at[b])

    def wait_store(k, b):
        pltpu.make_async_copy(sbuf.at[b],
                              out_hbm.at[c_hi, pl.ds(t0(k), TB),
                                         pl.ds(c_lo, 4)],
                              sem_s.at[b]).wait()

    pltpu.async_copy(yg_hbm.at[grp], tab, sem_t)
    fire_idx(0, 0)
    fire_idx(1, 1)
    pltpu.make_async_copy(yg_hbm.at[grp], tab, sem_t).wait()

    c1_vec = [jnp.full((16,), c, dtype=jnp.int32) for c in range(4)]
    c2_vec = [jnp.full((16,), 4 + c, dtype=jnp.int32) for c in range(4)]

    def body(k, _):
        b3 = lax.rem(k, 3)
        b2 = lax.rem(k, 2)

        wait_idx(k, b3)

        @pl.when(k + 2 < kmax)
        def _():
            fire_idx(k + 2, lax.rem(k + 2, 3))

        @pl.when(k >= 2)
        def _():
            wait_store(k - 2, b2)

        for t in range(TB):
            for g16 in range(TE // 16):
                src_v = idxb[b3, 0, pl.ds(t * TE + 16 * g16, 16)]
                dst_v = idxb[b3, 1, pl.ds(t * TE + 16 * g16, 16)]
                for c in range(4):
                    v = (plsc.load_gather(tab, [src_v, c1_vec[c]])
                         + plsc.load_gather(tab, [dst_v, c2_vec[c]]))
                    sbuf[b2, t, c, pl.ds(16 * g16, 16)] = v

        store(k, b2)
        return 0

    lax.fori_loop(0, kmax, body, 0)

    wait_store(kmax - 2, lax.rem(kmax - 2, 2))
    wait_store(kmax - 1, lax.rem(kmax - 1, 2))


def kernel(x, edge_index, W, b):
    yg = pl.pallas_call(
        _tables_kernel,
        out_shape=jax.ShapeDtypeStruct((NGRP, N, 8), jnp.float32),
    )(x, W, b.reshape(1, C))

    ei = edge_index.astype(jnp.int32)

    mesh = plsc.VectorSubcoreMesh(core_axis_name="c", subcore_axis_name="s",
                                  num_cores=NC, num_subcores=NS)
    out4 = pl.kernel(
        _edge_kernel,
        out_type=jax.ShapeDtypeStruct((2, NTILE, 8, TE), jnp.float32),
        mesh=mesh,
        scratch_types=[
            pltpu.VMEM((N, 8), jnp.float32),
            pltpu.VMEM((3, 2, TB * TE), jnp.int32),
            pltpu.VMEM((2, TB, 4, TE), jnp.float32),
            pltpu.SemaphoreType.DMA,
            pltpu.SemaphoreType.DMA((3,)),
            pltpu.SemaphoreType.DMA((2,)),
        ],
        compiler_params=pltpu.CompilerParams(use_tc_tiling_on_sc=False,
                                             needs_layout_passes=False),
    )(yg, ei)

    return out4.transpose(0, 2, 1, 3).reshape(C, E).T
